# baseline (speedup 1.0000x reference)
import functools

import jax
import jax.numpy as jnp
from jax import lax
from jax.experimental import pallas as pl
from jax.experimental.pallas import tpu as pltpu
from jax.experimental.pallas import tpu_sc as plsc

N_ROWS = 1000000
EMB = 16
HALF = 8
BATCH = 16384
LANES = 128

NUM_CORES = 2
NUM_SUBCORES = 16
NUM_WORKERS = NUM_CORES * NUM_SUBCORES
BPW = BATCH // NUM_WORKERS
GROUP = 16
NGROUPS = BPW // GROUP


def _bpr_body(uidx_hbm, pidx_hbm, nidx_hbm, ut_hbm, it_hbm,
              rui_hbm, ruj_hbm, loss_hbm,
              uidx_v, pidx_v, nidx_v,
              u_s0, p_s0, n_s0, u_s1, p_s1, n_s1,
              rui_v, ruj_v, loss_v, sem0, sem1):
    wid = lax.axis_index("s") * NUM_CORES + lax.axis_index("c")
    base = wid * BPW

    pltpu.sync_copy(uidx_hbm.at[pl.ds(base, BPW)], uidx_v)
    pltpu.sync_copy(pidx_hbm.at[pl.ds(base, BPW)], pidx_v)
    pltpu.sync_copy(nidx_hbm.at[pl.ds(base, BPW)], nidx_v)

    slots = ((u_s0, p_s0, n_s0, sem0), (u_s1, p_s1, n_s1, sem1))

    def load_iv(g):
        goff = pl.ds(g * GROUP, GROUP)
        return uidx_v[goff], pidx_v[goff], nidx_v[goff]

    def fire(g, half, slot):
        u_s, p_s, n_s, sem = slots[slot]
        iv_u, iv_p, iv_n = load_iv(g)
        blk_u = iv_u & ~(LANES - 1)
        blk_p = iv_p & ~(LANES - 1)
        blk_n = iv_n & ~(LANES - 1)
        rsrc = pl.ds(half * HALF, HALF)
        handles = []
        for i in range(GROUP):
            dst = pl.ds(i * HALF, HALF)
            bu = pl.multiple_of(blk_u[i], LANES)
            bp = pl.multiple_of(blk_p[i], LANES)
            bn = pl.multiple_of(blk_n[i], LANES)
            handles.append(pltpu.async_copy(
                ut_hbm.at[rsrc, pl.ds(bu, LANES)], u_s.at[dst], sem))
            handles.append(pltpu.async_copy(
                it_hbm.at[rsrc, pl.ds(bp, LANES)], p_s.at[dst], sem))
            handles.append(pltpu.async_copy(
                it_hbm.at[rsrc, pl.ds(bn, LANES)], n_s.at[dst], sem))
        return handles

    def drain(slot):
        u_s, p_s, n_s, sem = slots[slot]
        dummy = ut_hbm.at[pl.ds(0, HALF), pl.ds(0, LANES)]
        for i in range(GROUP):
            dst = pl.ds(i * HALF, HALF)
            pltpu.make_async_copy(dummy, u_s.at[dst], sem).wait()
            pltpu.make_async_copy(dummy, p_s.at[dst], sem).wait()
            pltpu.make_async_copy(dummy, n_s.at[dst], sem).wait()

    item_rows = lax.iota(jnp.int32, GROUP) * HALF

    def compute_half(g, slot, acc_ui, acc_uj, loss_acc):
        u_s, p_s, n_s, _ = slots[slot]
        iv_u, iv_p, iv_n = load_iv(g)
        lane_u = iv_u & (LANES - 1)
        lane_p = iv_p & (LANES - 1)
        lane_n = iv_n & (LANES - 1)
        for el in range(HALF):
            rows = item_rows + el
            u = plsc.load_gather(u_s, [rows, lane_u])
            p = plsc.load_gather(p_s, [rows, lane_p])
            n = plsc.load_gather(n_s, [rows, lane_n])
            acc_ui = acc_ui + u * p
            acc_uj = acc_uj + u * n
            loss_acc = loss_acc + (u * u + p * p + n * n)
        return acc_ui, acc_uj, loss_acc

    def group(k, loss_acc):
        h1 = fire(k, 1, 1)
        drain(0)
        zero = jnp.zeros((GROUP,), jnp.float32)
        acc_ui, acc_uj, loss_acc = compute_half(k, 0, zero, zero, loss_acc)

        @pl.when(k < NGROUPS - 1)
        def _():
            fire(k + 1, 0, 0)

        for cp in h1:
            cp.wait()
        acc_ui, acc_uj, loss_acc = compute_half(k, 1, acc_ui, acc_uj, loss_acc)
        goff = pl.ds(k * GROUP, GROUP)
        rui_v[goff] = acc_ui
        ruj_v[goff] = acc_uj
        return loss_acc

    fire(0, 0, 0)
    loss_acc = lax.fori_loop(0, NGROUPS, group, jnp.zeros((GROUP,), jnp.float32))
    loss_v[...] = loss_acc

    pltpu.sync_copy(rui_v, rui_hbm.at[pl.ds(base, BPW)])
    pltpu.sync_copy(ruj_v, ruj_hbm.at[pl.ds(base, BPW)])
    pltpu.sync_copy(loss_v, loss_hbm.at[wid])


@jax.jit
def _bpr_sc(uidx, pidx, nidx, ut, it):
    mesh = plsc.VectorSubcoreMesh(core_axis_name="c", subcore_axis_name="s")
    kern = functools.partial(
        pl.kernel,
        mesh=mesh,
        compiler_params=pltpu.CompilerParams(needs_layout_passes=False),
        out_type=[
            jax.ShapeDtypeStruct((BATCH,), jnp.float32),
            jax.ShapeDtypeStruct((BATCH,), jnp.float32),
            jax.ShapeDtypeStruct((NUM_WORKERS, EMB), jnp.float32),
        ],
        scratch_types=[
            pltpu.VMEM((BPW,), jnp.int32),
            pltpu.VMEM((BPW,), jnp.int32),
            pltpu.VMEM((BPW,), jnp.int32),
            pltpu.VMEM((GROUP * HALF, LANES), jnp.float32),
            pltpu.VMEM((GROUP * HALF, LANES), jnp.float32),
            pltpu.VMEM((GROUP * HALF, LANES), jnp.float32),
            pltpu.VMEM((GROUP * HALF, LANES), jnp.float32),
            pltpu.VMEM((GROUP * HALF, LANES), jnp.float32),
            pltpu.VMEM((GROUP * HALF, LANES), jnp.float32),
            pltpu.VMEM((BPW,), jnp.float32),
            pltpu.VMEM((BPW,), jnp.float32),
            pltpu.VMEM((EMB,), jnp.float32),
            pltpu.SemaphoreType.DMA,
            pltpu.SemaphoreType.DMA,
        ],
    )(_bpr_body)
    return kern(uidx, pidx, nidx, ut, it)


def kernel(users, pos_items, neg_items, user_emb, item_emb):
    users = users.astype(jnp.int32)
    pos_items = pos_items.astype(jnp.int32)
    neg_items = neg_items.astype(jnp.int32)
    ut = user_emb.T
    it = item_emb.T
    rui, ruj, loss_parts = _bpr_sc(users, pos_items, neg_items, ut, it)
    return (rui.reshape(BATCH, 1), ruj.reshape(BATCH, 1),
            jnp.sum(loss_parts))
